# baseline (speedup 1.0000x reference)
import functools

import jax
import jax.numpy as jnp
from jax.experimental import pallas as pl
from jax.experimental.pallas import tpu as pltpu
from jax.experimental.pallas import tpu_sc as plsc

_NC = 2
_NS = 16
_NW = _NC * _NS
_K = 125
_NBUF = 4
_NBUF3 = 4


def _pad_rows(n):
    return 128 * ((n + 127) // 128)


def _seg_sum_sc(g, em, zeros, width):
    npad = zeros.shape[0]
    nch = em.shape[2] // _NC
    rpt = npad // _NS
    mesh = plsc.VectorSubcoreMesh(core_axis_name="c", subcore_axis_name="s")

    @functools.partial(
        pl.kernel,
        out_type=jax.ShapeDtypeStruct((_NC, npad, width), jnp.float32),
        mesh=mesh,
        compiler_params=pltpu.CompilerParams(use_tc_tiling_on_sc=False),
        scratch_types=[
            pltpu.VMEM((nch, _K), jnp.int32),
            pltpu.VMEM((nch, _K), jnp.int32),
            pltpu.VMEM((_NBUF3, _K, width), jnp.float32),
        ] + [pltpu.SemaphoreType.DMA] * (2 * _NBUF3) + [
            pltpu.VMEM_SHARED((npad, width), jnp.float32),
        ],
    )
    def run(g_hbm, em_hbm, z_hbm, out_hbm, src_v, dst_v, rows_v,
            *rest):
        gsem = rest[:_NBUF3]
        ssem = rest[_NBUF3:2 * _NBUF3]
        acc_sh = rest[2 * _NBUF3]
        c = jax.lax.axis_index("c")
        s = jax.lax.axis_index("s")
        cb = pl.multiple_of(c * nch, 8)
        pltpu.sync_copy(em_hbm.at[0, s, pl.ds(cb, nch)], src_v)
        pltpu.sync_copy(em_hbm.at[1, s, pl.ds(cb, nch)], dst_v)
        pltpu.sync_copy(z_hbm.at[pl.ds(s * rpt, rpt)], acc_sh.at[pl.ds(s * rpt, rpt)])
        plsc.subcore_barrier()

        for b in range(_NBUF3):
            pltpu.async_copy(g_hbm.at[src_v.at[b]], rows_v.at[b], gsem[b])

        @pl.loop(0, nch, step=_NBUF3)
        def _(j0):
            for b in range(_NBUF3):
                pltpu.make_async_copy(g_hbm.at[src_v.at[b]], rows_v.at[b],
                                      gsem[b]).wait()
                pltpu.async_copy(rows_v.at[b], acc_sh.at[dst_v.at[j0 + b]],
                                 ssem[b], add=True)
            for b in range(_NBUF3):
                pltpu.make_async_copy(rows_v.at[b],
                                      acc_sh.at[dst_v.at[j0 + b]],
                                      ssem[b]).wait()

                @pl.when(j0 + b + _NBUF3 < nch)
                def _():
                    pltpu.async_copy(g_hbm.at[src_v.at[j0 + b + _NBUF3]],
                                     rows_v.at[b], gsem[b])

        plsc.subcore_barrier()
        pltpu.sync_copy(acc_sh.at[pl.ds(s * rpt, rpt)],
                        out_hbm.at[c, pl.ds(s * rpt, rpt)])

    return run(g, em, zeros)


def _deg_sc(em, ones, zeros):
    npad = zeros.shape[0]
    nch = em.shape[2] // _NC
    rpt = npad // _NS
    mesh = plsc.VectorSubcoreMesh(core_axis_name="c", subcore_axis_name="s")

    @functools.partial(
        pl.kernel,
        out_type=jax.ShapeDtypeStruct((_NC, npad, 8), jnp.float32),
        mesh=mesh,
        compiler_params=pltpu.CompilerParams(use_tc_tiling_on_sc=False),
        scratch_types=[
            pltpu.VMEM((nch, _K), jnp.int32),
            pltpu.VMEM((_K, 8), jnp.float32),
            pltpu.SemaphoreType.DMA,
            pltpu.VMEM_SHARED((npad, 8), jnp.float32),
        ],
    )
    def run(em_hbm, ones_hbm, z_hbm, out_hbm, dst_v, ones_v, sem, acc_sh):
        c = jax.lax.axis_index("c")
        s = jax.lax.axis_index("s")
        cb = pl.multiple_of(c * nch, 8)
        pltpu.sync_copy(em_hbm.at[1, s, pl.ds(cb, nch)], dst_v)
        pltpu.sync_copy(ones_hbm, ones_v)
        pltpu.sync_copy(z_hbm.at[pl.ds(s * rpt, rpt)], acc_sh.at[pl.ds(s * rpt, rpt)])
        plsc.subcore_barrier()

        @pl.loop(0, nch, step=8)
        def _(j0):
            for i in range(8):
                pltpu.async_copy(ones_v, acc_sh.at[dst_v.at[j0 + i]], sem,
                                 add=True)
            for i in range(8):
                pltpu.make_async_copy(ones_v, acc_sh.at[dst_v.at[j0 + i]],
                                      sem).wait()

        plsc.subcore_barrier()
        pltpu.sync_copy(acc_sh.at[pl.ds(s * rpt, rpt)],
                        out_hbm.at[c, pl.ds(s * rpt, rpt)])

    return run(em, ones, zeros)


def _gcn_fused_sc(g0h, em, dinv32, b1, zeros):
    npad = zeros.shape[0]
    nch = em.shape[2]
    hw = 32
    rpt = npad // _NS
    rsub = rpt // 4
    mesh = plsc.VectorSubcoreMesh(core_axis_name="c", subcore_axis_name="s")

    @functools.partial(
        pl.kernel,
        out_type=(jax.ShapeDtypeStruct((_NC, npad, hw), jnp.float32),
                  jax.ShapeDtypeStruct((_NC, npad, hw), jnp.float32)),
        mesh=mesh,
        compiler_params=pltpu.CompilerParams(use_tc_tiling_on_sc=False),
        scratch_types=[
            pltpu.VMEM((nch, _K), jnp.int32),
            pltpu.VMEM((nch, _K), jnp.int32),
            pltpu.VMEM((_NBUF, _K, hw), jnp.float32),
            pltpu.VMEM((rsub, hw), jnp.float32),
            pltpu.VMEM((rsub, hw), jnp.float32),
            pltpu.VMEM((rsub, hw), jnp.float32),
            pltpu.VMEM((hw,), jnp.float32),
        ] + [pltpu.SemaphoreType.DMA] * (2 * _NBUF) + [
            pltpu.VMEM_SHARED((npad, hw), jnp.float32),
            pltpu.VMEM_SHARED((npad, hw), jnp.float32),
        ],
    )
    def run(g0h_hbm, em_hbm, dinv_hbm, b1_hbm, z_hbm,
            s2u_hbm, u1h_hbm, src_v, dst_v, rows_v, buf_a, buf_b, buf_c,
            b1c, *rest):
        gsem = rest[:_NBUF]
        ssem = rest[_NBUF:2 * _NBUF]
        acc1 = rest[2 * _NBUF]
        acc2 = rest[2 * _NBUF + 1]
        c = jax.lax.axis_index("c")
        s = jax.lax.axis_index("s")
        pltpu.sync_copy(em_hbm.at[0, s], src_v)
        pltpu.sync_copy(em_hbm.at[1, s], dst_v)
        pltpu.sync_copy(b1_hbm.at[pl.ds(pl.multiple_of(c * hw, hw), hw)], b1c)
        pltpu.sync_copy(z_hbm.at[pl.ds(s * rpt, rpt)], acc1.at[pl.ds(s * rpt, rpt)])
        pltpu.sync_copy(z_hbm.at[pl.ds(s * rpt, rpt)], acc2.at[pl.ds(s * rpt, rpt)])
        plsc.subcore_barrier()

        def ring_pass(src_ref, acc_ref):
            for b in range(_NBUF):
                pltpu.async_copy(src_ref.at[src_v.at[b]], rows_v.at[b], gsem[b])

            @pl.loop(0, nch, step=_NBUF)
            def _(j0):
                for b in range(_NBUF):
                    pltpu.make_async_copy(src_ref.at[src_v.at[b]],
                                          rows_v.at[b], gsem[b]).wait()
                    pltpu.async_copy(rows_v.at[b], acc_ref.at[dst_v.at[j0 + b]],
                                     ssem[b], add=True)
                for b in range(_NBUF):
                    pltpu.make_async_copy(rows_v.at[b],
                                          acc_ref.at[dst_v.at[j0 + b]],
                                          ssem[b]).wait()

                    @pl.when(j0 + b + _NBUF < nch)
                    def _():
                        pltpu.async_copy(src_ref.at[src_v.at[j0 + b + _NBUF]],
                                         rows_v.at[b], gsem[b])

        ring_pass(g0h_hbm.at[c], acc1)
        plsc.subcore_barrier()

        bv0 = b1c[pl.ds(0, 16)]
        bv1 = b1c[pl.ds(16, 16)]
        for sub in range(4):
            r0 = s * rpt + sub * rsub
            pltpu.sync_copy(acc1.at[pl.ds(r0, rsub)], buf_a)
            pltpu.sync_copy(g0h_hbm.at[c, pl.ds(r0, rsub)], buf_b)
            pltpu.sync_copy(dinv_hbm.at[pl.ds(r0, rsub)], buf_c)

            @pl.loop(0, rsub)
            def _(r):
                for h, bv in ((0, bv0), (1, bv1)):
                    sl = pl.ds(h * 16, 16)
                    dv = buf_c[r, sl]
                    u = jnp.maximum(dv * (buf_a[r, sl] + buf_b[r, sl]) + bv,
                                    0.0) * dv
                    buf_a[r, sl] = u

            pltpu.sync_copy(buf_a, u1h_hbm.at[c, pl.ds(r0, rsub)])
        plsc.subcore_barrier()

        ring_pass(u1h_hbm.at[c], acc2)
        plsc.subcore_barrier()
        pltpu.sync_copy(acc2.at[pl.ds(s * rpt, rpt)],
                        s2u_hbm.at[c, pl.ds(s * rpt, rpt)])

    return run(g0h, em, dinv32, b1, zeros)


def _tc1a_body(x_ref, w1_ref, h0_ref):
    n = x_ref.shape[0]
    h0_ref[0:n, :] = jnp.dot(x_ref[...], w1_ref[...],
                             preferred_element_type=jnp.float32)


def _tc1_body(h0_ref, degp_ref, g0h_ref, dinv32_ref):
    nb = h0_ref.shape[0]
    deg = degp_ref[0, :, 0:1] + degp_ref[1, :, 0:1] + 1.0
    dinv = jax.lax.rsqrt(deg)
    g0 = h0_ref[...] * dinv
    g0h_ref[0] = g0[:, 0:32]
    g0h_ref[1] = g0[:, 32:64]
    dinv32_ref[...] = jnp.broadcast_to(dinv, (nb, 32))


def _tc3_body(s2u_ref, u1h_ref, dinv32_ref, b2_ref, w2_ref, g2_ref):
    nb = g2_ref.shape[0]
    dinv = dinv32_ref[:, 0:1]
    va = s2u_ref[0] + u1h_ref[0]
    vb = s2u_ref[1] + u1h_ref[1]
    h2 = dinv * (jnp.dot(va, w2_ref[0:32], preferred_element_type=jnp.float32)
                 + jnp.dot(vb, w2_ref[32:64],
                           preferred_element_type=jnp.float32)) + b2_ref[...]
    sq = jnp.sum(h2 * h2, axis=1, keepdims=True)
    g2_ref[:, 0:64] = h2
    g2_ref[:, 64:65] = sq
    g2_ref[:, 65:72] = jnp.zeros((nb, 7), jnp.float32)


def _tc4_body(s3p_ref, g2_ref, degp_ref, t2_ref, tf_ref, wlin_ref, blin_ref,
              out_ref):
    nt = tf_ref.shape[0]
    ntn = tf_ref.shape[1]
    deg = degp_ref[0, :, 0:1] + degp_ref[1, :, 0:1] + 1.0
    s3 = s3p_ref[0] + s3p_ref[1] + g2_ref[...]
    mean = s3 / deg
    mean_x = mean[:, 0:64]
    mean_sq = mean[:, 64:65]

    mf = tf_ref[:, 0, :]
    mfsq = jnp.sum(tf_ref[:, 0, :] ** 2, axis=1)
    for k in range(1, ntn):
        tk = tf_ref[:, k, :]
        mf = mf + tk
        mfsq = mfsq + jnp.sum(tk * tk, axis=1)
    mf = mf * (1.0 / ntn)
    mfsq = (mfsq * (1.0 / ntn)).reshape(1, nt)
    struct = (jnp.sum(t2_ref[...] ** 2, axis=1) / t2_ref.shape[1]).reshape(1, nt)

    cross = jax.lax.dot_general(mean_x, mf, (((1,), (1,)), ((), ())),
                                preferred_element_type=jnp.float32)
    feat = mean_sq + mfsq - 2.0 * cross
    y = 0.5 * feat + 0.5 * struct

    h2 = g2_ref[:, 0:64]
    out = (jnp.dot(h2, wlin_ref[0:64], preferred_element_type=jnp.float32)
           + jnp.dot(y, wlin_ref[64:80], preferred_element_type=jnp.float32)
           + blin_ref[...])
    out_ref[...] = jnp.maximum(out, 0.0)


def _tc_call(body, out_shapes, *args):
    return pl.pallas_call(
        body,
        out_shape=[jax.ShapeDtypeStruct(s, jnp.float32) for s in out_shapes],
    )(*args)


def kernel(x, edge_index, W1, b1, W2, b2, templates, templates_features,
           W_lin, b_lin):
    n = x.shape[0]
    e = edge_index.shape[1]
    nt = templates.shape[0]
    nch2 = e // (_NS * _K)

    em = edge_index.reshape(2, _NS, nch2, _K)
    npad = _pad_rows(n)
    ones8 = jnp.ones((_K, 8), jnp.float32)
    z8 = jnp.zeros((npad, 8), jnp.float32)
    z32 = jnp.zeros((npad, 32), jnp.float32)
    z72 = jnp.zeros((npad, 72), jnp.float32)

    nblk = 8
    rb = npad // nblk
    full = lambda shape: pl.BlockSpec(shape, lambda i: tuple(0 for _ in shape))

    degp = _deg_sc(em, ones8, z8)
    (h0,) = _tc_call(_tc1a_body, [(npad, 64)], x, W1)
    g0h, dinv32 = pl.pallas_call(
        _tc1_body,
        grid=(nblk,),
        in_specs=[pl.BlockSpec((rb, 64), lambda i: (i, 0)),
                  pl.BlockSpec((2, rb, 8), lambda i: (0, i, 0))],
        out_specs=[pl.BlockSpec((2, rb, 32), lambda i: (0, i, 0)),
                   pl.BlockSpec((rb, 32), lambda i: (i, 0))],
        out_shape=[jax.ShapeDtypeStruct((2, npad, 32), jnp.float32),
                   jax.ShapeDtypeStruct((npad, 32), jnp.float32)],
    )(h0, degp)
    s2u, u1h = _gcn_fused_sc(g0h, em, dinv32, b1, z32)
    (g2,) = pl.pallas_call(
        _tc3_body,
        grid=(nblk,),
        in_specs=[pl.BlockSpec((2, rb, 32), lambda i: (0, i, 0)),
                  pl.BlockSpec((2, rb, 32), lambda i: (0, i, 0)),
                  pl.BlockSpec((rb, 32), lambda i: (i, 0)),
                  full((64,)), full((64, 64))],
        out_specs=[pl.BlockSpec((rb, 72), lambda i: (i, 0))],
        out_shape=[jax.ShapeDtypeStruct((npad, 72), jnp.float32)],
    )(s2u, u1h, dinv32, b2, W2)
    s3p = _seg_sum_sc(g2, em, z72, 72)
    (outp,) = pl.pallas_call(
        _tc4_body,
        grid=(nblk,),
        in_specs=[pl.BlockSpec((2, rb, 72), lambda i: (0, i, 0)),
                  pl.BlockSpec((rb, 72), lambda i: (i, 0)),
                  pl.BlockSpec((2, rb, 8), lambda i: (0, i, 0)),
                  full((nt, templates.shape[1] * templates.shape[2])),
                  full(templates_features.shape),
                  full(W_lin.shape), full(b_lin.shape)],
        out_specs=[pl.BlockSpec((rb, W_lin.shape[1]), lambda i: (i, 0))],
        out_shape=[jax.ShapeDtypeStruct((npad, W_lin.shape[1]), jnp.float32)],
    )(s3p, g2, degp, templates.reshape(nt, -1), templates_features,
      W_lin, b_lin)
    return outp[:n]


# speedup vs baseline: 1.0028x; 1.0028x over previous
import functools

import jax
import jax.numpy as jnp
from jax.experimental import pallas as pl
from jax.experimental.pallas import tpu as pltpu
from jax.experimental.pallas import tpu_sc as plsc

_NC = 2
_NS = 16
_NW = _NC * _NS
_K = 125
_NBUF = 4
_NBUF3 = 4


def _pad_rows(n):
    return 128 * ((n + 127) // 128)


def _seg_sum_sc(g, em, zeros, width):
    npad = zeros.shape[0]
    nch = em.shape[2] // _NC
    rpt = npad // _NS
    mesh = plsc.VectorSubcoreMesh(core_axis_name="c", subcore_axis_name="s")

    @functools.partial(
        pl.kernel,
        out_type=jax.ShapeDtypeStruct((_NC, npad, width), jnp.float32),
        mesh=mesh,
        compiler_params=pltpu.CompilerParams(use_tc_tiling_on_sc=False),
        scratch_types=[
            pltpu.VMEM((nch, _K), jnp.int32),
            pltpu.VMEM((nch, _K), jnp.int32),
            pltpu.VMEM((_NBUF3, _K, width), jnp.float32),
        ] + [pltpu.SemaphoreType.DMA] * (2 * _NBUF3) + [
            pltpu.VMEM_SHARED((npad, width), jnp.float32),
        ],
    )
    def run(g_hbm, em_hbm, z_hbm, out_hbm, src_v, dst_v, rows_v,
            *rest):
        gsem = rest[:_NBUF3]
        ssem = rest[_NBUF3:2 * _NBUF3]
        acc_sh = rest[2 * _NBUF3]
        c = jax.lax.axis_index("c")
        s = jax.lax.axis_index("s")
        cb = pl.multiple_of(c * nch, 8)
        pltpu.sync_copy(em_hbm.at[0, s, pl.ds(cb, nch)], src_v)
        pltpu.sync_copy(em_hbm.at[1, s, pl.ds(cb, nch)], dst_v)
        pltpu.sync_copy(z_hbm.at[pl.ds(s * rpt, rpt)], acc_sh.at[pl.ds(s * rpt, rpt)])
        plsc.subcore_barrier()

        for b in range(_NBUF3):
            pltpu.async_copy(g_hbm.at[src_v.at[b]], rows_v.at[b], gsem[b])

        @pl.loop(0, nch, step=_NBUF3)
        def _(j0):
            for b in range(_NBUF3):
                pltpu.make_async_copy(g_hbm.at[src_v.at[b]], rows_v.at[b],
                                      gsem[b]).wait()
                pltpu.async_copy(rows_v.at[b], acc_sh.at[dst_v.at[j0 + b]],
                                 ssem[b], add=True)
            for b in range(_NBUF3):
                pltpu.make_async_copy(rows_v.at[b],
                                      acc_sh.at[dst_v.at[j0 + b]],
                                      ssem[b]).wait()

                @pl.when(j0 + b + _NBUF3 < nch)
                def _():
                    pltpu.async_copy(g_hbm.at[src_v.at[j0 + b + _NBUF3]],
                                     rows_v.at[b], gsem[b])

        plsc.subcore_barrier()
        pltpu.sync_copy(acc_sh.at[pl.ds(s * rpt, rpt)],
                        out_hbm.at[c, pl.ds(s * rpt, rpt)])

    return run(g, em, zeros)


def _deg_sc(em, ones, zeros):
    npad = zeros.shape[0]
    nch = em.shape[2] // _NC
    rpt = npad // _NS
    mesh = plsc.VectorSubcoreMesh(core_axis_name="c", subcore_axis_name="s")

    @functools.partial(
        pl.kernel,
        out_type=jax.ShapeDtypeStruct((_NC, npad, 8), jnp.float32),
        mesh=mesh,
        compiler_params=pltpu.CompilerParams(use_tc_tiling_on_sc=False),
        scratch_types=[
            pltpu.VMEM((nch, _K), jnp.int32),
            pltpu.VMEM((_K, 8), jnp.float32),
            pltpu.SemaphoreType.DMA,
            pltpu.VMEM_SHARED((npad, 8), jnp.float32),
        ],
    )
    def run(em_hbm, ones_hbm, z_hbm, out_hbm, dst_v, ones_v, sem, acc_sh):
        c = jax.lax.axis_index("c")
        s = jax.lax.axis_index("s")
        cb = pl.multiple_of(c * nch, 8)
        pltpu.sync_copy(em_hbm.at[1, s, pl.ds(cb, nch)], dst_v)
        pltpu.sync_copy(ones_hbm, ones_v)
        pltpu.sync_copy(z_hbm.at[pl.ds(s * rpt, rpt)], acc_sh.at[pl.ds(s * rpt, rpt)])
        plsc.subcore_barrier()

        @pl.loop(0, nch, step=8)
        def _(j0):
            for i in range(8):
                pltpu.async_copy(ones_v, acc_sh.at[dst_v.at[j0 + i]], sem,
                                 add=True)
            for i in range(8):
                pltpu.make_async_copy(ones_v, acc_sh.at[dst_v.at[j0 + i]],
                                      sem).wait()

        plsc.subcore_barrier()
        pltpu.sync_copy(acc_sh.at[pl.ds(s * rpt, rpt)],
                        out_hbm.at[c, pl.ds(s * rpt, rpt)])

    return run(em, ones, zeros)


def _gcn_fused_sc(g0h, em, dinv32, b1, zeros):
    npad = zeros.shape[0]
    nch = em.shape[2]
    hw = 32
    rpt = npad // _NS
    rsub = rpt // 4
    mesh = plsc.VectorSubcoreMesh(core_axis_name="c", subcore_axis_name="s")

    @functools.partial(
        pl.kernel,
        out_type=(jax.ShapeDtypeStruct((_NC, npad, hw), jnp.float32),
                  jax.ShapeDtypeStruct((_NC, npad, hw), jnp.float32)),
        mesh=mesh,
        compiler_params=pltpu.CompilerParams(use_tc_tiling_on_sc=False),
        scratch_types=[
            pltpu.VMEM((nch, _K), jnp.int32),
            pltpu.VMEM((nch, _K), jnp.int32),
            pltpu.VMEM((_NBUF, _K, hw), jnp.float32),
            pltpu.VMEM((rsub, hw), jnp.float32),
            pltpu.VMEM((rsub, hw), jnp.float32),
            pltpu.VMEM((rsub, hw), jnp.float32),
            pltpu.VMEM((hw,), jnp.float32),
        ] + [pltpu.SemaphoreType.DMA] * (2 * _NBUF) + [
            pltpu.VMEM_SHARED((npad, hw), jnp.float32),
            pltpu.VMEM_SHARED((npad, hw), jnp.float32),
        ],
    )
    def run(g0h_hbm, em_hbm, dinv_hbm, b1_hbm, z_hbm,
            s2u_hbm, u1h_hbm, src_v, dst_v, rows_v, buf_a, buf_b, buf_c,
            b1c, *rest):
        gsem = rest[:_NBUF]
        ssem = rest[_NBUF:2 * _NBUF]
        acc1 = rest[2 * _NBUF]
        acc2 = rest[2 * _NBUF + 1]
        c = jax.lax.axis_index("c")
        s = jax.lax.axis_index("s")
        pltpu.sync_copy(em_hbm.at[0, s], src_v)
        pltpu.sync_copy(em_hbm.at[1, s], dst_v)
        pltpu.sync_copy(b1_hbm.at[pl.ds(pl.multiple_of(c * hw, hw), hw)], b1c)
        pltpu.sync_copy(z_hbm.at[pl.ds(s * rpt, rpt)], acc1.at[pl.ds(s * rpt, rpt)])
        pltpu.sync_copy(z_hbm.at[pl.ds(s * rpt, rpt)], acc2.at[pl.ds(s * rpt, rpt)])
        plsc.subcore_barrier()

        def ring_pass(src_ref, acc_ref):
            for b in range(_NBUF):
                pltpu.async_copy(src_ref.at[src_v.at[b]], rows_v.at[b], gsem[b])

            @pl.loop(0, nch, step=_NBUF)
            def _(j0):
                for b in range(_NBUF):
                    pltpu.make_async_copy(src_ref.at[src_v.at[b]],
                                          rows_v.at[b], gsem[b]).wait()
                    pltpu.async_copy(rows_v.at[b], acc_ref.at[dst_v.at[j0 + b]],
                                     ssem[b], add=True)
                for b in range(_NBUF):
                    pltpu.make_async_copy(rows_v.at[b],
                                          acc_ref.at[dst_v.at[j0 + b]],
                                          ssem[b]).wait()

                    @pl.when(j0 + b + _NBUF < nch)
                    def _():
                        pltpu.async_copy(src_ref.at[src_v.at[j0 + b + _NBUF]],
                                         rows_v.at[b], gsem[b])

        ring_pass(g0h_hbm.at[c], acc1)
        plsc.subcore_barrier()

        bv0 = b1c[pl.ds(0, 16)]
        bv1 = b1c[pl.ds(16, 16)]
        for sub in range(4):
            r0 = s * rpt + sub * rsub
            pltpu.sync_copy(acc1.at[pl.ds(r0, rsub)], buf_a)
            pltpu.sync_copy(g0h_hbm.at[c, pl.ds(r0, rsub)], buf_b)
            pltpu.sync_copy(dinv_hbm.at[pl.ds(r0, rsub)], buf_c)

            @pl.loop(0, rsub)
            def _(r):
                for h, bv in ((0, bv0), (1, bv1)):
                    sl = pl.ds(h * 16, 16)
                    dv = buf_c[r, sl]
                    u = jnp.maximum(dv * (buf_a[r, sl] + buf_b[r, sl]) + bv,
                                    0.0) * dv
                    buf_a[r, sl] = u

            pltpu.sync_copy(buf_a, u1h_hbm.at[c, pl.ds(r0, rsub)])
        plsc.subcore_barrier()

        ring_pass(u1h_hbm.at[c], acc2)
        plsc.subcore_barrier()
        pltpu.sync_copy(acc2.at[pl.ds(s * rpt, rpt)],
                        s2u_hbm.at[c, pl.ds(s * rpt, rpt)])

    return run(g0h, em, dinv32, b1, zeros)


def _tc1a_body(x_ref, w1_ref, h0_ref):
    n = x_ref.shape[0]
    h0_ref[0:n, :] = jnp.dot(x_ref[...], w1_ref[...],
                             preferred_element_type=jnp.float32)


def _tc1_body(h0_ref, degp_ref, g0h_ref, dinv32_ref):
    nb = h0_ref.shape[0]
    deg = degp_ref[0, :, 0:1] + degp_ref[1, :, 0:1] + 1.0
    dinv = jax.lax.rsqrt(deg)
    g0 = h0_ref[...] * dinv
    g0h_ref[0] = g0[:, 0:32]
    g0h_ref[1] = g0[:, 32:64]
    dinv32_ref[...] = jnp.broadcast_to(dinv, (nb, 32))


def _tc3_body(s2u_ref, u1h_ref, dinv32_ref, b2_ref, w2_ref, g2_ref):
    nb = g2_ref.shape[0]
    dinv = dinv32_ref[:, 0:1]
    va = s2u_ref[0] + u1h_ref[0]
    vb = s2u_ref[1] + u1h_ref[1]
    h2 = dinv * (jnp.dot(va, w2_ref[0:32], preferred_element_type=jnp.float32)
                 + jnp.dot(vb, w2_ref[32:64],
                           preferred_element_type=jnp.float32)) + b2_ref[...]
    sq = jnp.sum(h2 * h2, axis=1, keepdims=True)
    g2_ref[:, 0:64] = h2
    g2_ref[:, 64:65] = sq
    g2_ref[:, 65:72] = jnp.zeros((nb, 7), jnp.float32)


def _tc4_body(s3p_ref, g2_ref, dinv32_ref, t2_ref, tf_ref, wlin_ref, blin_ref,
              out_ref):
    nt = tf_ref.shape[0]
    ntn = tf_ref.shape[1]
    dinv = dinv32_ref[:, 0:1]
    s3 = s3p_ref[0] + s3p_ref[1] + g2_ref[...]
    mean = s3 * (dinv * dinv)
    mean_x = mean[:, 0:64]
    mean_sq = mean[:, 64:65]

    mf = tf_ref[:, 0, :]
    mfsq = jnp.sum(tf_ref[:, 0, :] ** 2, axis=1)
    for k in range(1, ntn):
        tk = tf_ref[:, k, :]
        mf = mf + tk
        mfsq = mfsq + jnp.sum(tk * tk, axis=1)
    mf = mf * (1.0 / ntn)
    mfsq = (mfsq * (1.0 / ntn)).reshape(1, nt)
    struct = (jnp.sum(t2_ref[...] ** 2, axis=1) / t2_ref.shape[1]).reshape(1, nt)

    cross = jax.lax.dot_general(mean_x, mf, (((1,), (1,)), ((), ())),
                                preferred_element_type=jnp.float32)
    feat = mean_sq + mfsq - 2.0 * cross
    y = 0.5 * feat + 0.5 * struct

    h2 = g2_ref[:, 0:64]
    out = (jnp.dot(h2, wlin_ref[0:64], preferred_element_type=jnp.float32)
           + jnp.dot(y, wlin_ref[64:80], preferred_element_type=jnp.float32)
           + blin_ref[...])
    out_ref[...] = jnp.maximum(out, 0.0)


def _tc_call(body, out_shapes, *args):
    return pl.pallas_call(
        body,
        out_shape=[jax.ShapeDtypeStruct(s, jnp.float32) for s in out_shapes],
    )(*args)


def kernel(x, edge_index, W1, b1, W2, b2, templates, templates_features,
           W_lin, b_lin):
    n = x.shape[0]
    e = edge_index.shape[1]
    nt = templates.shape[0]
    nch2 = e // (_NS * _K)

    em = edge_index.reshape(2, _NS, nch2, _K)
    npad = _pad_rows(n)
    ones8 = jnp.ones((_K, 8), jnp.float32)
    z8 = jnp.zeros((npad, 8), jnp.float32)
    z32 = jnp.zeros((npad, 32), jnp.float32)
    z72 = jnp.zeros((npad, 72), jnp.float32)

    nblk = 8
    rb = npad // nblk
    full = lambda shape: pl.BlockSpec(shape, lambda i: tuple(0 for _ in shape))

    degp = _deg_sc(em, ones8, z8)
    (h0,) = _tc_call(_tc1a_body, [(npad, 64)], x, W1)
    g0h, dinv32 = pl.pallas_call(
        _tc1_body,
        grid=(nblk,),
        in_specs=[pl.BlockSpec((rb, 64), lambda i: (i, 0)),
                  pl.BlockSpec((2, rb, 8), lambda i: (0, i, 0))],
        out_specs=[pl.BlockSpec((2, rb, 32), lambda i: (0, i, 0)),
                   pl.BlockSpec((rb, 32), lambda i: (i, 0))],
        out_shape=[jax.ShapeDtypeStruct((2, npad, 32), jnp.float32),
                   jax.ShapeDtypeStruct((npad, 32), jnp.float32)],
    )(h0, degp)
    s2u, u1h = _gcn_fused_sc(g0h, em, dinv32, b1, z32)
    (g2,) = pl.pallas_call(
        _tc3_body,
        grid=(nblk,),
        in_specs=[pl.BlockSpec((2, rb, 32), lambda i: (0, i, 0)),
                  pl.BlockSpec((2, rb, 32), lambda i: (0, i, 0)),
                  pl.BlockSpec((rb, 32), lambda i: (i, 0)),
                  full((64,)), full((64, 64))],
        out_specs=[pl.BlockSpec((rb, 72), lambda i: (i, 0))],
        out_shape=[jax.ShapeDtypeStruct((npad, 72), jnp.float32)],
    )(s2u, u1h, dinv32, b2, W2)
    s3p = _seg_sum_sc(g2, em, z72, 72)
    (outp,) = pl.pallas_call(
        _tc4_body,
        grid=(nblk,),
        in_specs=[pl.BlockSpec((2, rb, 72), lambda i: (0, i, 0)),
                  pl.BlockSpec((rb, 72), lambda i: (i, 0)),
                  pl.BlockSpec((rb, 32), lambda i: (i, 0)),
                  full((nt, templates.shape[1] * templates.shape[2])),
                  full(templates_features.shape),
                  full(W_lin.shape), full(b_lin.shape)],
        out_specs=[pl.BlockSpec((rb, W_lin.shape[1]), lambda i: (i, 0))],
        out_shape=[jax.ShapeDtypeStruct((npad, W_lin.shape[1]), jnp.float32)],
    )(s3p, g2, dinv32, templates.reshape(nt, -1), templates_features,
      W_lin, b_lin)
    return outp[:n]
